# baseline (speedup 1.0000x reference)
import functools

import jax
import jax.numpy as jnp
from jax import lax
from jax.experimental import pallas as pl
from jax.experimental.pallas import tpu as pltpu
from jax.experimental.pallas import tpu_sc as plsc

_NC = 2
_NS = 16
_NW = _NC * _NS
_NBUF = 5
_LA = 3
_SPLIT = 1


@functools.lru_cache(maxsize=None)
def _make_gather(BS, T, V, D):
    assert BS % (_NW * _SPLIT) == 0
    b_chunk = BS // (_NW * _SPLIT)
    n_chunks = T * _SPLIT
    assert n_chunks % _NBUF == 0
    n_rounds = n_chunks // _NBUF
    mesh = plsc.VectorSubcoreMesh(core_axis_name="c", subcore_axis_name="s")

    @functools.partial(
        pl.kernel,
        mesh=mesh,
        out_type=jax.ShapeDtypeStruct((T, BS, D), jnp.float32),
        scratch_types=[
            pltpu.VMEM((T, b_chunk * _SPLIT), jnp.int32),
            pltpu.VMEM((_NBUF, b_chunk, D), jnp.float32),
            [pltpu.SemaphoreType.DMA] * _NBUF,
            [pltpu.SemaphoreType.DMA] * _NBUF,
        ],
    )
    def gather_kernel(table_hbm, idx_hbm, out_hbm, idx_v, rows_v, gsems, wsems):
        wid = lax.axis_index("s") * _NC + lax.axis_index("c")
        base_b = wid * b_chunk * _SPLIT
        t_head = 8
        pltpu.sync_copy(
            idx_hbm.at[pl.ds(0, t_head), pl.ds(base_b, b_chunk * _SPLIT)],
            idx_v.at[pl.ds(0, t_head)],
        )

        def issue_gather(c, j, h):
            pltpu.async_copy(
                table_hbm.at[idx_v.at[c // _SPLIT, pl.ds(h * b_chunk, b_chunk)]],
                rows_v.at[j],
                gsems[j],
            )

        def wait_gather(j):
            pltpu.make_async_copy(
                table_hbm.at[idx_v.at[0, pl.ds(0, b_chunk)]], rows_v.at[j], gsems[j]
            ).wait()

        def issue_write(c, j, h):
            pltpu.async_copy(
                rows_v.at[j],
                out_hbm.at[c // _SPLIT, pl.ds(base_b + h * b_chunk, b_chunk)],
                wsems[j],
            )

        def wait_write(j):
            pltpu.make_async_copy(
                rows_v.at[j], out_hbm.at[0, pl.ds(base_b, b_chunk)], wsems[j]
            ).wait()

        assert (_LA - 1) // _SPLIT < t_head
        for j in range(_LA):
            issue_gather(j, j, j % _SPLIT)
        pltpu.sync_copy(
            idx_hbm.at[pl.ds(t_head, T - t_head), pl.ds(base_b, b_chunk * _SPLIT)],
            idx_v.at[pl.ds(t_head, T - t_head)],
        )

        def consume_prefetch(r, j, first_round):
            c = r * _NBUF + j
            wait_gather(j)
            issue_write(c, j, j % _SPLIT)
            cp = c + _LA
            jp = (j + _LA) % _NBUF
            if not (first_round and j < _NBUF - _LA):
                wait_write(jp)
            issue_gather(cp, jp, (j + _LA) % _SPLIT)

        for j in range(_NBUF):
            consume_prefetch(0, j, True)

        def round_body(r, _):
            for j in range(_NBUF):
                consume_prefetch(r, j, False)
            return ()

        lax.fori_loop(1, n_rounds - 1, round_body, (), unroll=False)

        last = (n_rounds - 1) * _NBUF
        for j in range(_NBUF):
            c = last + j
            wait_gather(j)
            issue_write(c, j, j % _SPLIT)
            if j < _NBUF - _LA:
                jp = (j + _LA) % _NBUF
                wait_write(jp)
                issue_gather(c + _LA, jp, (j + _LA) % _SPLIT)
        for j in range(_NBUF):
            wait_write(j)

    return gather_kernel


def kernel(x, table):
    bs, seq = x.shape
    V, D = table.shape
    xt = x.astype(jnp.int32).T
    out = _make_gather(bs, seq, V, D)(table, xt)
    return out.transpose(1, 0, 2)


# speedup vs baseline: 1.0016x; 1.0016x over previous
import functools

import jax
import jax.numpy as jnp
from jax import lax
from jax.experimental import pallas as pl
from jax.experimental.pallas import tpu as pltpu
from jax.experimental.pallas import tpu_sc as plsc

_NC = 2
_NS = 16
_NW = _NC * _NS
_NBUF = 10
_LA = 7
_SPLIT = 2


@functools.lru_cache(maxsize=None)
def _make_gather(BS, T, V, D):
    assert BS % (_NW * _SPLIT) == 0
    b_chunk = BS // (_NW * _SPLIT)
    n_chunks = T * _SPLIT
    assert n_chunks % _NBUF == 0
    n_rounds = n_chunks // _NBUF
    mesh = plsc.VectorSubcoreMesh(core_axis_name="c", subcore_axis_name="s")

    @functools.partial(
        pl.kernel,
        mesh=mesh,
        out_type=jax.ShapeDtypeStruct((T, BS, D), jnp.float32),
        scratch_types=[
            pltpu.VMEM((T, b_chunk * _SPLIT), jnp.int32),
            pltpu.VMEM((_NBUF, b_chunk, D), jnp.float32),
            [pltpu.SemaphoreType.DMA] * _NBUF,
            [pltpu.SemaphoreType.DMA] * _NBUF,
        ],
    )
    def gather_kernel(table_hbm, idx_hbm, out_hbm, idx_v, rows_v, gsems, wsems):
        wid = lax.axis_index("s") * _NC + lax.axis_index("c")
        base_b = wid * b_chunk * _SPLIT
        t_head = 8
        pltpu.sync_copy(
            idx_hbm.at[pl.ds(0, t_head), pl.ds(base_b, b_chunk * _SPLIT)],
            idx_v.at[pl.ds(0, t_head)],
        )

        def issue_gather(c, j, h):
            pltpu.async_copy(
                table_hbm.at[idx_v.at[c // _SPLIT, pl.ds(h * b_chunk, b_chunk)]],
                rows_v.at[j],
                gsems[j],
            )

        def wait_gather(j):
            pltpu.make_async_copy(
                table_hbm.at[idx_v.at[0, pl.ds(0, b_chunk)]], rows_v.at[j], gsems[j]
            ).wait()

        def issue_write(c, j, h):
            pltpu.async_copy(
                rows_v.at[j],
                out_hbm.at[c // _SPLIT, pl.ds(base_b + h * b_chunk, b_chunk)],
                wsems[j],
            )

        def wait_write(j):
            pltpu.make_async_copy(
                rows_v.at[j], out_hbm.at[0, pl.ds(base_b, b_chunk)], wsems[j]
            ).wait()

        assert (_LA - 1) // _SPLIT < t_head
        for j in range(_LA):
            issue_gather(j, j, j % _SPLIT)
        pltpu.sync_copy(
            idx_hbm.at[pl.ds(t_head, T - t_head), pl.ds(base_b, b_chunk * _SPLIT)],
            idx_v.at[pl.ds(t_head, T - t_head)],
        )

        def consume_prefetch(r, j, first_round):
            c = r * _NBUF + j
            wait_gather(j)
            issue_write(c, j, j % _SPLIT)
            cp = c + _LA
            jp = (j + _LA) % _NBUF
            if not (first_round and j < _NBUF - _LA):
                wait_write(jp)
            issue_gather(cp, jp, (j + _LA) % _SPLIT)

        for j in range(_NBUF):
            consume_prefetch(0, j, True)

        def round_body(r, _):
            for j in range(_NBUF):
                consume_prefetch(r, j, False)
            return ()

        lax.fori_loop(1, n_rounds - 1, round_body, (), unroll=False)

        last = (n_rounds - 1) * _NBUF
        for j in range(_NBUF):
            c = last + j
            wait_gather(j)
            issue_write(c, j, j % _SPLIT)
            if j < _NBUF - _LA:
                jp = (j + _LA) % _NBUF
                wait_write(jp)
                issue_gather(c + _LA, jp, (j + _LA) % _SPLIT)
        for j in range(_NBUF):
            wait_write(j)

    return gather_kernel


def kernel(x, table):
    bs, seq = x.shape
    V, D = table.shape
    xt = x.astype(jnp.int32).T
    out = _make_gather(bs, seq, V, D)(table, xt)
    return out.transpose(1, 0, 2)
